# baseline (speedup 1.0000x reference)
import functools

import jax
import jax.numpy as jnp
from jax import lax
from jax.experimental import pallas as pl
from jax.experimental.pallas import tpu as pltpu
from jax.experimental.pallas import tpu_sc as plsc

C = 32
D = 512
N = 250
K = 4
SIGMA = 0.05
NPAD = 256
HIST = D * K
HISTP = HIST + 16
L = 16


def _tc_body(xr_ref, xc_ref, noise_ref, gidx_ref, rank_ref):
    x_row = xr_ref[0]
    x_col = xc_ref[0]
    ii = lax.broadcasted_iota(jnp.int32, (D, D), 0)
    jj = lax.broadcasted_iota(jnp.int32, (D, D), 1)

    m = (x_row > x_col) | ((x_row == x_col) & (jj < ii))
    rank_col = jnp.sum(m.astype(jnp.int32), axis=1, keepdims=True)
    rank_ref[0] = rank_col

    a2 = (rank_col == jj).astype(jnp.float32)
    x_sorted_row = jnp.sum(a2 * x_col, axis=0, keepdims=True)

    noisy = noise_ref[0] * SIGMA + x_sorted_row
    iota_d = lax.broadcasted_iota(jnp.int32, (N, D), 1)
    neg = jnp.float32(-jnp.inf)
    idxs = []
    for _ in range(K):
        mx = jnp.max(noisy, axis=1, keepdims=True)
        amx = jnp.min(jnp.where(noisy == mx, iota_d, D), axis=1, keepdims=True)
        idxs.append(amx)
        noisy = jnp.where(iota_d == amx, neg, noisy)

    a, b, c, d = idxs
    a, b = jnp.minimum(a, b), jnp.maximum(a, b)
    c, d = jnp.minimum(c, d), jnp.maximum(c, d)
    a, c = jnp.minimum(a, c), jnp.maximum(a, c)
    b, d = jnp.minimum(b, d), jnp.maximum(b, d)
    b, c = jnp.minimum(b, c), jnp.maximum(b, c)

    pad = jnp.full((NPAD - N, K), HIST, jnp.int32)
    flat = jnp.concatenate(
        [a * K + 0, b * K + 1, c * K + 2, d * K + 3], axis=1)
    gidx_ref[0] = jnp.concatenate([flat, pad], axis=0)


def _tc_stage(x, noise):
    return pl.pallas_call(
        _tc_body,
        grid=(C,),
        in_specs=[
            pl.BlockSpec((1, 1, D), lambda c: (c, 0, 0)),
            pl.BlockSpec((1, D, 1), lambda c: (c, 0, 0)),
            pl.BlockSpec((1, N, D), lambda c: (c, 0, 0)),
        ],
        out_specs=[
            pl.BlockSpec((1, NPAD, K), lambda c: (c, 0, 0)),
            pl.BlockSpec((1, D, 1), lambda c: (c, 0, 0)),
        ],
        out_shape=[
            jax.ShapeDtypeStruct((C, NPAD, K), jnp.int32),
            jax.ShapeDtypeStruct((C, D, 1), jnp.int32),
        ],
    )(x.reshape(C, 1, D), x.reshape(C, D, 1), noise)


def _sc_kernel(gidx_hbm, rank_hbm, y_hbm, sel_v, rank_v, hist_v, out_v, sem):
    del sem
    nc = plsc.get_sparse_core_info().num_cores
    cid = lax.axis_index("s") * nc + lax.axis_index("c")
    pltpu.sync_copy(gidx_hbm.at[cid], sel_v)
    pltpu.sync_copy(rank_hbm.at[cid], rank_v)

    lanes = lax.broadcasted_iota(jnp.int32, (L,), 0)
    zeros = jnp.zeros((L,), jnp.float32)
    ones = jnp.ones((L,), jnp.float32)

    def zero_body(i, _):
        hist_v[pl.ds(i * L, L)] = zeros
        return 0
    lax.fori_loop(0, HISTP // L, zero_body, 0)

    def scat_body(i, _):
        plsc.addupdate_scatter(hist_v, [sel_v[pl.ds(i * L, L)]], ones)
        return 0
    lax.fori_loop(0, (NPAD * K) // L, scat_body, 0)

    inv_n = jnp.float32(1.0 / N)
    jrep = lanes // K
    krep = lanes % K

    def gat_body(i, _):
        r = plsc.load_gather(rank_v, [i * (L // K) + jrep])
        vals = plsc.load_gather(hist_v, [r * K + krep])
        out_v[pl.ds(i * L, L)] = vals * inv_n
        return 0
    lax.fori_loop(0, (D * K) // L, gat_body, 0)

    pltpu.sync_copy(out_v, y_hbm.at[cid])


def _sc_stage(gidx, rank):
    mesh = plsc.VectorSubcoreMesh(core_axis_name="c", subcore_axis_name="s")
    run = pl.kernel(
        _sc_kernel,
        out_type=jax.ShapeDtypeStruct((C, D * K), jnp.float32),
        mesh=mesh,
        scratch_types=[
            pltpu.VMEM((NPAD * K,), jnp.int32),
            pltpu.VMEM((D,), jnp.int32),
            pltpu.VMEM((HISTP,), jnp.float32),
            pltpu.VMEM((D * K,), jnp.float32),
            pltpu.SemaphoreType.DMA,
        ],
        compiler_params=pltpu.CompilerParams(needs_layout_passes=False),
    )
    return run(gidx.reshape(C, NPAD * K), rank.reshape(C, D))


def kernel(x, noise):
    gidx, rank = _tc_stage(x, noise)
    y = _sc_stage(gidx, rank)
    return y.reshape(C, D, K)


# speedup vs baseline: 1.1680x; 1.1680x over previous
import functools

import jax
import jax.numpy as jnp
from jax import lax
from jax.experimental import pallas as pl
from jax.experimental.pallas import tpu as pltpu
from jax.experimental.pallas import tpu_sc as plsc

C = 32
D = 512
N = 250
K = 4
SIGMA = 0.05
NPAD = 256
HIST = D * K
HISTP = HIST + 16
L = 16


def _tc_body(xr_ref, xc_ref, noise_hbm, gidx_ref, rank_ref, nbuf, sems):
    cstep = pl.program_id(0)
    slot = lax.rem(cstep, 2)
    nxt = lax.rem(cstep + 1, 2)

    @pl.when(cstep == 0)
    def _():
        pltpu.make_async_copy(noise_hbm.at[0], nbuf.at[0], sems.at[0]).start()

    @pl.when(cstep + 1 < C)
    def _():
        pltpu.make_async_copy(
            noise_hbm.at[cstep + 1], nbuf.at[nxt], sems.at[nxt]).start()

    x_row = xr_ref[0]
    x_col = xc_ref[0]
    ii = lax.broadcasted_iota(jnp.int32, (D, D), 0)
    jj = lax.broadcasted_iota(jnp.int32, (D, D), 1)

    m = (x_row > x_col) | ((x_row == x_col) & (jj < ii))
    rank_col = jnp.sum(m.astype(jnp.int32), axis=1, keepdims=True)
    rank_ref[0] = rank_col

    a2 = (rank_col == jj).astype(jnp.float32)
    x_sorted_row = jnp.sum(a2 * x_col, axis=0, keepdims=True)

    pltpu.make_async_copy(
        noise_hbm.at[cstep], nbuf.at[slot], sems.at[slot]).wait()
    noisy = nbuf[slot] * SIGMA + x_sorted_row
    neg = jnp.float32(-jnp.inf)
    iota_f = lax.broadcasted_iota(jnp.int32, (N, D), 1).astype(jnp.float32)
    idxs = []
    for _ in range(K):
        mx = jnp.max(noisy, axis=1, keepdims=True)
        eq = noisy == mx
        amxf = jnp.sum(eq.astype(jnp.float32) * iota_f,
                       axis=1, keepdims=True)
        idxs.append(amxf.astype(jnp.int32))
        noisy = jnp.where(eq, neg, noisy)

    a, b, c, d = idxs
    a, b = jnp.minimum(a, b), jnp.maximum(a, b)
    c, d = jnp.minimum(c, d), jnp.maximum(c, d)
    a, c = jnp.minimum(a, c), jnp.maximum(a, c)
    b, d = jnp.minimum(b, d), jnp.maximum(b, d)
    b, c = jnp.minimum(b, c), jnp.maximum(b, c)

    pad = jnp.full((NPAD - N, K), HIST, jnp.int32)
    flat = jnp.concatenate(
        [a * K + 0, b * K + 1, c * K + 2, d * K + 3], axis=1)
    gidx_ref[0] = jnp.concatenate([flat, pad], axis=0)


def _tc_stage(x, noise):
    return pl.pallas_call(
        _tc_body,
        grid=(C,),
        in_specs=[
            pl.BlockSpec((1, 1, D), lambda c: (c, 0, 0)),
            pl.BlockSpec((1, D, 1), lambda c: (c, 0, 0)),
            pl.BlockSpec(memory_space=pl.ANY),
        ],
        scratch_shapes=[
            pltpu.VMEM((2, N, D), jnp.float32),
            pltpu.SemaphoreType.DMA((2,)),
        ],
        out_specs=[
            pl.BlockSpec((1, NPAD, K), lambda c: (c, 0, 0)),
            pl.BlockSpec((1, D, 1), lambda c: (c, 0, 0)),
        ],
        out_shape=[
            jax.ShapeDtypeStruct((C, NPAD, K), jnp.int32),
            jax.ShapeDtypeStruct((C, D, 1), jnp.int32),
        ],
    )(x.reshape(C, 1, D), x.reshape(C, D, 1), noise)


def _sc_kernel(gidx_hbm, rank_hbm, y_hbm, sel_v, rank_v, hist_v, out_v, sem):
    del sem
    nc = plsc.get_sparse_core_info().num_cores
    cid = lax.axis_index("s") * nc + lax.axis_index("c")
    pltpu.sync_copy(gidx_hbm.at[cid], sel_v)
    pltpu.sync_copy(rank_hbm.at[cid], rank_v)

    lanes = lax.broadcasted_iota(jnp.int32, (L,), 0)
    zeros = jnp.zeros((L,), jnp.float32)
    ones = jnp.ones((L,), jnp.float32)

    def zero_body(i, _):
        hist_v[pl.ds(i * L, L)] = zeros
        return 0
    lax.fori_loop(0, HISTP // L, zero_body, 0)

    def scat_body(i, _):
        plsc.addupdate_scatter(hist_v, [sel_v[pl.ds(i * L, L)]], ones)
        return 0
    lax.fori_loop(0, (NPAD * K) // L, scat_body, 0)

    inv_n = jnp.float32(1.0 / N)
    jrep = lanes // K
    krep = lanes % K

    def gat_body(i, _):
        r = plsc.load_gather(rank_v, [i * (L // K) + jrep])
        vals = plsc.load_gather(hist_v, [r * K + krep])
        out_v[pl.ds(i * L, L)] = vals * inv_n
        return 0
    lax.fori_loop(0, (D * K) // L, gat_body, 0)

    pltpu.sync_copy(out_v, y_hbm.at[cid])


def _sc_stage(gidx, rank):
    mesh = plsc.VectorSubcoreMesh(core_axis_name="c", subcore_axis_name="s")
    run = pl.kernel(
        _sc_kernel,
        out_type=jax.ShapeDtypeStruct((C, D * K), jnp.float32),
        mesh=mesh,
        scratch_types=[
            pltpu.VMEM((NPAD * K,), jnp.int32),
            pltpu.VMEM((D,), jnp.int32),
            pltpu.VMEM((HISTP,), jnp.float32),
            pltpu.VMEM((D * K,), jnp.float32),
            pltpu.SemaphoreType.DMA,
        ],
        compiler_params=pltpu.CompilerParams(needs_layout_passes=False),
    )
    return run(gidx.reshape(C, NPAD * K), rank.reshape(C, D))


def kernel(x, noise):
    gidx, rank = _tc_stage(x, noise)
    y = _sc_stage(gidx, rank)
    return y.reshape(C, D, K)
